# baseline (speedup 1.0000x reference)
import functools

import jax
import jax.numpy as jnp
from jax import lax
from jax.experimental import pallas as pl
from jax.experimental.pallas import tpu as pltpu
from jax.experimental.pallas import tpu_sc as plsc

N_GRID = 10000
N_MESH = 10000
E = 320000
D = 128
H = 128

NC = 2
NS = 16
NW = NC * NS
EPW = E // NW
CHUNK = 80
NCHUNK = EPW // CHUNK


def _ln(y, g, b):
    m = jnp.mean(y, axis=-1, keepdims=True)
    v = jnp.mean((y - m) ** 2, axis=-1, keepdims=True)
    return (y - m) * lax.rsqrt(v + 1e-5) * g + b


def _silu(x):
    return x * jax.nn.sigmoid(x)


def _s1a_body(grid_ref, mesh_ref, w0b_ref, w0c_ref, eb0_ref, tbl_ref):
    tbl_ref[0] = grid_ref[...] @ w0b_ref[...] + eb0_ref[...]
    tbl_ref[1] = mesh_ref[...] @ w0c_ref[...]


def _stage1a(grid_nfeat, mesh_nfeat, w0b, w0c, eb0):
    R = 1000
    row = pl.BlockSpec((R, D), lambda i: (i, 0))
    mat = pl.BlockSpec((D, H), lambda i: (0, 0))
    vec = pl.BlockSpec((1, H), lambda i: (0, 0))
    return pl.pallas_call(
        _s1a_body,
        grid=(N_GRID // R,),
        in_specs=[row, row, mat, mat, vec],
        out_specs=pl.BlockSpec((NC, R, D), lambda i: (0, i, 0)),
        out_shape=jax.ShapeDtypeStruct((NC, N_GRID, H), jnp.float32),
    )(grid_nfeat, mesh_nfeat, w0b, w0c, eb0)


def _s1b_body(grid_ref, sw0_ref, sb0_ref, sw1_ref, sb1_ref, sg_ref, sbt_ref,
              gout_ref):
    g = grid_ref[...]
    h = _silu(g @ sw0_ref[...] + sb0_ref[...])
    y = h @ sw1_ref[...] + sb1_ref[...]
    gout_ref[...] = g + _ln(y, sg_ref[...], sbt_ref[...])


def _stage1b(grid_nfeat, sw0, sb0, sw1, sb1, sg, sbt):
    R = 1000
    row = pl.BlockSpec((R, D), lambda i: (i, 0))
    mat = pl.BlockSpec((D, H), lambda i: (0, 0))
    vec = pl.BlockSpec((1, H), lambda i: (0, 0))
    return pl.pallas_call(
        _s1b_body,
        grid=(N_GRID // R,),
        in_specs=[row, mat, vec, mat, vec, vec, vec],
        out_specs=row,
        out_shape=jax.ShapeDtypeStruct((N_GRID, D), jnp.float32),
    )(grid_nfeat, sw0, sb0, sw1, sb1, sg, sbt)


_RING = 4


def _gather_body(epw, nchunk, pg_hbm, pm_hbm, src_hbm, dst_hbm, out_hbm,
                 idxs_v, idxd_v, bas, bbs, gas, gbs, wss):
    c = lax.axis_index("c")
    s = lax.axis_index("s")
    w = s * NC + c
    base = w * epw
    pltpu.sync_copy(src_hbm.at[w], idxs_v)
    pltpu.sync_copy(dst_hbm.at[w], idxd_v)

    def issue_gather(j, b):
        pltpu.async_copy(pg_hbm.at[idxs_v.at[j]], bas[b], gas[b])
        pltpu.async_copy(pm_hbm.at[idxd_v.at[j]], bbs[b], gbs[b])

    def wait_gather(j, b):
        pltpu.make_async_copy(pg_hbm.at[idxs_v.at[j]], bas[b], gas[b]).wait()
        pltpu.make_async_copy(pm_hbm.at[idxd_v.at[j]], bbs[b], gbs[b]).wait()

    def add_bufs(b):
        ba, bb = bas[b], bbs[b]

        def add_row(r, carry2):
            for q in range(D // 16):
                sl = pl.ds(q * 16, 16)
                ba[r, sl] = ba[r, sl] + bb[r, sl]
            return carry2

        lax.fori_loop(0, CHUNK, add_row, 0, unroll=2)

    def issue_write(j, b):
        pltpu.async_copy(bas[b], out_hbm.at[pl.ds(base + j * CHUNK, CHUNK)],
                         wss[b])

    def wait_write(j, b):
        pltpu.make_async_copy(
            bas[b], out_hbm.at[pl.ds(base + j * CHUNK, CHUNK)], wss[b]).wait()

    def step(j, b):
        wait_gather(j, b)
        add_bufs(b)
        issue_write(j, b)

    for b in range(_RING):
        issue_gather(b, b)
    step(0, 0)
    step(1, 1)

    def body(i, carry):
        for b in range(_RING):
            j = 2 + 4 * i + b
            step(j, (2 + b) % 4)
            wait_write(j - 2, b)
            issue_gather(j + 2, b)
        return carry

    lax.fori_loop(0, (nchunk - 5) // 4, body, 0, unroll=False)
    jt = nchunk - 3
    step(jt, jt % 4)
    wait_write(jt - 2, (jt - 2) % 4)
    issue_gather(jt + 2, (jt + 2) % 4)
    step(jt + 1, (jt + 1) % 4)
    wait_write(jt - 1, (jt - 1) % 4)
    step(jt + 2, (jt + 2) % 4)
    wait_write(jt, jt % 4)
    wait_write(jt + 1, (jt + 1) % 4)
    wait_write(jt + 2, (jt + 2) % 4)


_GC = 80


def _gather_spmem_body(ept, nchunk, tbl_hbm, idx_hbm, out_hbm,
                       tbl_sh, idx_v, buf0, buf1, g0, g1, w0, w1):
    c = lax.axis_index("c")
    s = lax.axis_index("s")
    base = s * ept

    @pl.when(s < 15)
    def _():
        pltpu.sync_copy(tbl_hbm.at[c, pl.ds(s * 640, 640)],
                        tbl_sh.at[pl.ds(s * 640, 640)])

    @pl.when(s == 15)
    def _():
        pltpu.sync_copy(tbl_hbm.at[c, pl.ds(9600, 400)],
                        tbl_sh.at[pl.ds(9600, 400)])

    plsc.subcore_barrier()
    pltpu.sync_copy(idx_hbm.at[c, s], idx_v)
    bufs = ((buf0, g0, w0), (buf1, g1, w1))

    def issue_gather(j, p):
        b, g, _ = bufs[p]
        pltpu.async_copy(tbl_sh.at[idx_v.at[j]], b, g)

    def wait_gather(j, p):
        b, g, _ = bufs[p]
        pltpu.make_async_copy(tbl_sh.at[idx_v.at[j]], b, g).wait()

    def issue_write(jo, p):
        b, _, w = bufs[p]
        pltpu.async_copy(b, out_hbm.at[c, pl.ds(base + jo * _GC, _GC)], w)

    def wait_write(jo, p):
        b, _, w = bufs[p]
        pltpu.make_async_copy(
            b, out_hbm.at[c, pl.ds(base + jo * _GC, _GC)], w).wait()

    npair = nchunk // 2
    issue_gather(0, 0)
    issue_gather(1, 1)

    def pair(i, carry):
        j1 = 2 * i
        j2 = 2 * i + 1
        wait_gather(j1, 0)
        issue_write(j1, 0)
        wait_gather(j2, 1)
        issue_write(j2, 1)
        wait_write(j1, 0)

        @pl.when(i < npair - 1)
        def _():
            issue_gather(j1 + 2, 0)

        wait_write(j2, 1)

        @pl.when(i < npair - 1)
        def _():
            issue_gather(j2 + 2, 1)

        return carry

    lax.fori_loop(0, npair, pair, 0, unroll=False)


def _stage2_spmem(tbl2, idx2, ne):
    ept = ne // NS
    nchunk = ept // _GC
    assert nchunk % 2 == 0
    mesh = plsc.VectorSubcoreMesh(core_axis_name="c", subcore_axis_name="s")
    fn = pl.kernel(
        functools.partial(_gather_spmem_body, ept, nchunk),
        out_type=jax.ShapeDtypeStruct((NC, ne, D), jnp.float32),
        mesh=mesh,
        scratch_types=[
            pltpu.VMEM_SHARED((N_MESH, D), jnp.float32),
            pltpu.VMEM((nchunk, _GC), jnp.int32),
            pltpu.VMEM((_GC, D), jnp.float32),
            pltpu.VMEM((_GC, D), jnp.float32),
            pltpu.SemaphoreType.DMA,
            pltpu.SemaphoreType.DMA,
            pltpu.SemaphoreType.DMA,
            pltpu.SemaphoreType.DMA,
        ],
    )
    return fn(tbl2, idx2)


def _stage2(pg, pm, src_r, dst_r, ne):
    epw = ne // NW
    nchunk = epw // CHUNK
    assert nchunk % 4 == 1 and nchunk >= 5
    mesh = plsc.VectorSubcoreMesh(core_axis_name="c", subcore_axis_name="s")
    fn = pl.kernel(
        functools.partial(_gather_body, epw, nchunk),
        out_type=jax.ShapeDtypeStruct((ne, D), jnp.float32),
        mesh=mesh,
        scratch_types=[
            pltpu.VMEM((nchunk, CHUNK), jnp.int32),
            pltpu.VMEM((nchunk, CHUNK), jnp.int32),
            [pltpu.VMEM((CHUNK, D), jnp.float32)] * _RING,
            [pltpu.VMEM((CHUNK, D), jnp.float32)] * _RING,
            [pltpu.SemaphoreType.DMA] * _RING,
            [pltpu.SemaphoreType.DMA] * _RING,
            [pltpu.SemaphoreType.DMA] * _RING,
        ],
    )
    return fn(pg, pm, src_r, dst_r)


def _edge_body(ef_ref, g2_ref, w0a_ref, w1_ref, eb1_ref, eg_ref, ebt_ref,
               out_ref):
    h = _silu(ef_ref[...] @ w0a_ref[...] + g2_ref[0] + g2_ref[1])
    y = h @ w1_ref[...] + eb1_ref[...]
    out_ref[...] = _ln(y, eg_ref[...], ebt_ref[...])


def _stage3(efeat, g2, w0a, w1, eb1, eg, ebt, blk0, ne):
    R = 1000
    erow = pl.BlockSpec((R, D), lambda i: (i + blk0, 0))
    row = pl.BlockSpec((R, D), lambda i: (i, 0))
    g2spec = pl.BlockSpec((NC, R, D), lambda i: (0, i, 0))
    mat = pl.BlockSpec((D, H), lambda i: (0, 0))
    vec = pl.BlockSpec((1, H), lambda i: (0, 0))
    return pl.pallas_call(
        _edge_body,
        grid=(ne // R,),
        in_specs=[erow, g2spec, mat, mat, vec, vec, vec],
        out_specs=row,
        out_shape=jax.ShapeDtypeStruct((ne, D), jnp.float32),
    )(efeat, g2, w0a, w1, eb1, eg, ebt)


def _scatter_body(epw, nchunk, nslice, *refs):
    y_hbms = refs[:nslice]
    (dst_hbm, zeros_hbm, out_hbm,
     idx_v, u0, u1, acc_sh, rs0, rs1, ss0, ss1) = refs[nslice:]
    c = lax.axis_index("c")
    s = lax.axis_index("s")
    w = s * NC + c
    base = w * epw

    @pl.when(s == 0)
    def _():
        pltpu.sync_copy(zeros_hbm, acc_sh)

    plsc.subcore_barrier()
    pltpu.sync_copy(dst_hbm.at[w], idx_v)
    bufs = ((u0, rs0, ss0), (u1, rs1, ss1))

    def run_slice(y_hbm, joff):
        def issue_read(j, p):
            u, rs, _ = bufs[p]
            pltpu.async_copy(y_hbm.at[pl.ds(base + j * CHUNK, CHUNK)], u, rs)

        def wait_read(j, p):
            u, rs, _ = bufs[p]
            pltpu.make_async_copy(
                y_hbm.at[pl.ds(base + j * CHUNK, CHUNK)], u, rs).wait()

        def issue_scatter(j, p):
            u, _, ss = bufs[p]
            pltpu.async_copy(u, acc_sh.at[idx_v.at[joff + j]], ss, add=True)

        def wait_scatter(j, p):
            u, _, ss = bufs[p]
            pltpu.make_async_copy(u, acc_sh.at[idx_v.at[joff + j]], ss).wait()

        npair = (nchunk - 1) // 2
        issue_read(0, 1)
        wait_read(0, 1)
        issue_scatter(0, 1)
        issue_read(1, 0)
        wait_scatter(0, 1)
        issue_read(2, 1)

        def pair(i, carry):
            j1 = 2 * i + 1
            j2 = 2 * i + 2
            wait_read(j1, 0)
            issue_scatter(j1, 0)
            wait_read(j2, 1)
            issue_scatter(j2, 1)
            wait_scatter(j1, 0)

            @pl.when(i < npair - 1)
            def _():
                issue_read(j1 + 2, 0)

            wait_scatter(j2, 1)

            @pl.when(i < npair - 1)
            def _():
                issue_read(j2 + 2, 1)

            return carry

        lax.fori_loop(0, npair, pair, 0, unroll=False)

    for si in range(nslice):
        run_slice(y_hbms[si], si * nchunk)
    plsc.subcore_barrier()

    @pl.when(s < 15)
    def _():
        pltpu.sync_copy(acc_sh.at[pl.ds(s * 640, 640)],
                        out_hbm.at[c, pl.ds(s * 640, 640)])

    @pl.when(s == 15)
    def _():
        pltpu.sync_copy(acc_sh.at[pl.ds(9600, 400)],
                        out_hbm.at[c, pl.ds(9600, 400)])


def _stage4(mlp_list, dst_r, zeros, ne):
    epw = ne // NW
    nchunk = epw // CHUNK
    nslice = len(mlp_list)
    mesh = plsc.VectorSubcoreMesh(core_axis_name="c", subcore_axis_name="s")
    fn = pl.kernel(
        functools.partial(_scatter_body, epw, nchunk, nslice),
        out_type=jax.ShapeDtypeStruct((NC, N_MESH, D), jnp.float32),
        mesh=mesh,
        scratch_types=[
            pltpu.VMEM((nslice * nchunk, CHUNK), jnp.int32),
            pltpu.VMEM((CHUNK, D), jnp.float32),
            pltpu.VMEM((CHUNK, D), jnp.float32),
            pltpu.VMEM_SHARED((N_MESH, D), jnp.float32),
            pltpu.SemaphoreType.DMA,
            pltpu.SemaphoreType.DMA,
            pltpu.SemaphoreType.DMA,
            pltpu.SemaphoreType.DMA,
        ],
    )
    return fn(*mlp_list, dst_r, zeros)


def _s5_body(*refs):
    nparts = len(refs) - 9
    parts = refs[:nparts]
    (mesh_ref, dw0a_ref, dw0b_ref, db0_ref,
     dw1_ref, db1_ref, dg_ref, dbt_ref, out_ref) = refs[nparts:]
    agg = parts[0][0] + parts[0][1]
    for p in parts[1:]:
        agg = agg + p[0] + p[1]
    m = mesh_ref[...]
    h = _silu(agg @ dw0a_ref[...] + m @ dw0b_ref[...] + db0_ref[...])
    y = h @ dw1_ref[...] + db1_ref[...]
    out_ref[...] = m + _ln(y, dg_ref[...], dbt_ref[...])


def _stage5(parts, mesh_nfeat, dw0a, dw0b, db0, dw1, db1, dg, dbt):
    R = 1000
    row = pl.BlockSpec((R, D), lambda i: (i, 0))
    mat = pl.BlockSpec((D, H), lambda i: (0, 0))
    vec = pl.BlockSpec((1, H), lambda i: (0, 0))
    pspec = pl.BlockSpec((NC, R, D), lambda i: (0, i, 0))
    return pl.pallas_call(
        _s5_body,
        grid=(N_MESH // R,),
        in_specs=[pspec] * len(parts) + [row, mat, mat, vec, mat, vec, vec, vec],
        out_specs=row,
        out_shape=jax.ShapeDtypeStruct((N_MESH, D), jnp.float32),
    )(*parts, mesh_nfeat, dw0a, dw0b, db0, dw1, db1, dg, dbt)


def kernel(g2m_efeat, grid_nfeat, mesh_nfeat,
           eW0, eb0, eW1, eb1, eg, ebt,
           sW0, sb0, sW1, sb1, sg, sbt,
           dW0, db0, dW1, db1, dg, dbt,
           src, dst):
    w0a, w0b, w0c = eW0[:D], eW0[D:2 * D], eW0[2 * D:]
    dw0a, dw0b = dW0[:D], dW0[D:]
    r2 = lambda v: v.reshape(1, -1)

    tbl2 = _stage1a(grid_nfeat, mesh_nfeat, w0b, w0c, r2(eb0))
    grid_out = _stage1b(grid_nfeat, sW0, r2(sb0), sW1, r2(sb1), r2(sg),
                        r2(sbt))

    S = 5
    ES = E // S
    nchunk = ES // NW // CHUNK
    dst_r = dst.reshape(S, NW, nchunk, CHUNK).transpose(1, 0, 2, 3).reshape(
        NW, S * nchunk, CHUNK)
    ngc = ES // NS // _GC
    idx_r = jnp.stack([src, dst]).reshape(2, S, NS, ngc, _GC).transpose(
        1, 0, 2, 3, 4)
    zeros = jnp.zeros((N_MESH, D), jnp.float32)

    mlps = []
    for i in range(S):
        g2 = _stage2_spmem(tbl2, idx_r[i], ES)
        mlps.append(_stage3(g2m_efeat, g2, w0a, eW1, r2(eb1), r2(eg), r2(ebt),
                            i * (ES // 1000), ES))
    part = _stage4(mlps, dst_r, zeros, ES)

    mesh_out = _stage5([part], mesh_nfeat, dw0a, dw0b, r2(db0),
                       dW1, r2(db1), r2(dg), r2(dbt))
    return (grid_out, mesh_out)


# speedup vs baseline: 1.1213x; 1.1213x over previous
import functools

import jax
import jax.numpy as jnp
from jax import lax
from jax.experimental import pallas as pl
from jax.experimental.pallas import tpu as pltpu
from jax.experimental.pallas import tpu_sc as plsc

N_GRID = 10000
N_MESH = 10000
E = 320000
D = 128
H = 128

NC = 2
NS = 16
NW = NC * NS
EPW = E // NW
CHUNK = 80
NCHUNK = EPW // CHUNK


def _ln(y, g, b):
    m = jnp.mean(y, axis=-1, keepdims=True)
    v = jnp.mean((y - m) ** 2, axis=-1, keepdims=True)
    return (y - m) * lax.rsqrt(v + 1e-5) * g + b


def _silu(x):
    return x * jax.nn.sigmoid(x)


def _s1a_body(grid_ref, mesh_ref, w0b_ref, w0c_ref, eb0_ref, tbl_ref):
    tbl_ref[0] = grid_ref[...] @ w0b_ref[...] + eb0_ref[...]
    tbl_ref[1] = mesh_ref[...] @ w0c_ref[...]


def _stage1a(grid_nfeat, mesh_nfeat, w0b, w0c, eb0):
    R = 1000
    row = pl.BlockSpec((R, D), lambda i: (i, 0))
    mat = pl.BlockSpec((D, H), lambda i: (0, 0))
    vec = pl.BlockSpec((1, H), lambda i: (0, 0))
    return pl.pallas_call(
        _s1a_body,
        grid=(N_GRID // R,),
        in_specs=[row, row, mat, mat, vec],
        out_specs=pl.BlockSpec((NC, R, D), lambda i: (0, i, 0)),
        out_shape=jax.ShapeDtypeStruct((NC, N_GRID, H), jnp.float32),
    )(grid_nfeat, mesh_nfeat, w0b, w0c, eb0)


def _s1b_body(grid_ref, sw0_ref, sb0_ref, sw1_ref, sb1_ref, sg_ref, sbt_ref,
              gout_ref):
    g = grid_ref[...]
    h = _silu(g @ sw0_ref[...] + sb0_ref[...])
    y = h @ sw1_ref[...] + sb1_ref[...]
    gout_ref[...] = g + _ln(y, sg_ref[...], sbt_ref[...])


def _stage1b(grid_nfeat, sw0, sb0, sw1, sb1, sg, sbt):
    R = 1000
    row = pl.BlockSpec((R, D), lambda i: (i, 0))
    mat = pl.BlockSpec((D, H), lambda i: (0, 0))
    vec = pl.BlockSpec((1, H), lambda i: (0, 0))
    return pl.pallas_call(
        _s1b_body,
        grid=(N_GRID // R,),
        in_specs=[row, mat, vec, mat, vec, vec, vec],
        out_specs=row,
        out_shape=jax.ShapeDtypeStruct((N_GRID, D), jnp.float32),
    )(grid_nfeat, sw0, sb0, sw1, sb1, sg, sbt)


_RING = 4


def _gather_body(epw, nchunk, pg_hbm, pm_hbm, src_hbm, dst_hbm, out_hbm,
                 idxs_v, idxd_v, bas, bbs, gas, gbs, wss):
    c = lax.axis_index("c")
    s = lax.axis_index("s")
    w = s * NC + c
    base = w * epw
    pltpu.sync_copy(src_hbm.at[w], idxs_v)
    pltpu.sync_copy(dst_hbm.at[w], idxd_v)

    def issue_gather(j, b):
        pltpu.async_copy(pg_hbm.at[idxs_v.at[j]], bas[b], gas[b])
        pltpu.async_copy(pm_hbm.at[idxd_v.at[j]], bbs[b], gbs[b])

    def wait_gather(j, b):
        pltpu.make_async_copy(pg_hbm.at[idxs_v.at[j]], bas[b], gas[b]).wait()
        pltpu.make_async_copy(pm_hbm.at[idxd_v.at[j]], bbs[b], gbs[b]).wait()

    def add_bufs(b):
        ba, bb = bas[b], bbs[b]

        def add_row(r, carry2):
            for q in range(D // 16):
                sl = pl.ds(q * 16, 16)
                ba[r, sl] = ba[r, sl] + bb[r, sl]
            return carry2

        lax.fori_loop(0, CHUNK, add_row, 0, unroll=2)

    def issue_write(j, b):
        pltpu.async_copy(bas[b], out_hbm.at[pl.ds(base + j * CHUNK, CHUNK)],
                         wss[b])

    def wait_write(j, b):
        pltpu.make_async_copy(
            bas[b], out_hbm.at[pl.ds(base + j * CHUNK, CHUNK)], wss[b]).wait()

    def step(j, b):
        wait_gather(j, b)
        add_bufs(b)
        issue_write(j, b)

    for b in range(_RING):
        issue_gather(b, b)
    step(0, 0)
    step(1, 1)

    def body(i, carry):
        for b in range(_RING):
            j = 2 + 4 * i + b
            step(j, (2 + b) % 4)
            wait_write(j - 2, b)
            issue_gather(j + 2, b)
        return carry

    lax.fori_loop(0, (nchunk - 5) // 4, body, 0, unroll=False)
    jt = nchunk - 3
    step(jt, jt % 4)
    wait_write(jt - 2, (jt - 2) % 4)
    issue_gather(jt + 2, (jt + 2) % 4)
    step(jt + 1, (jt + 1) % 4)
    wait_write(jt - 1, (jt - 1) % 4)
    step(jt + 2, (jt + 2) % 4)
    wait_write(jt, jt % 4)
    wait_write(jt + 1, (jt + 1) % 4)
    wait_write(jt + 2, (jt + 2) % 4)


_GC = 80


def _gather_spmem_body(ept, nchunk, tbl_hbm, idx_hbm, out_hbm,
                       tbl_sh, idx_v, buf0, buf1, g0, g1, w0, w1):
    c = lax.axis_index("c")
    s = lax.axis_index("s")
    base = s * ept

    @pl.when(s < 15)
    def _():
        pltpu.sync_copy(tbl_hbm.at[c, pl.ds(s * 640, 640)],
                        tbl_sh.at[pl.ds(s * 640, 640)])

    @pl.when(s == 15)
    def _():
        pltpu.sync_copy(tbl_hbm.at[c, pl.ds(9600, 400)],
                        tbl_sh.at[pl.ds(9600, 400)])

    plsc.subcore_barrier()
    pltpu.sync_copy(idx_hbm.at[c, s], idx_v)
    bufs = ((buf0, g0, w0), (buf1, g1, w1))

    def issue_gather(j, p):
        b, g, _ = bufs[p]
        pltpu.async_copy(tbl_sh.at[idx_v.at[j]], b, g)

    def wait_gather(j, p):
        b, g, _ = bufs[p]
        pltpu.make_async_copy(tbl_sh.at[idx_v.at[j]], b, g).wait()

    def issue_write(jo, p):
        b, _, w = bufs[p]
        pltpu.async_copy(b, out_hbm.at[c, pl.ds(base + jo * _GC, _GC)], w)

    def wait_write(jo, p):
        b, _, w = bufs[p]
        pltpu.make_async_copy(
            b, out_hbm.at[c, pl.ds(base + jo * _GC, _GC)], w).wait()

    npair = nchunk // 2
    issue_gather(0, 0)
    issue_gather(1, 1)

    def pair(i, carry):
        j1 = 2 * i
        j2 = 2 * i + 1
        wait_gather(j1, 0)
        issue_write(j1, 0)
        wait_gather(j2, 1)
        issue_write(j2, 1)
        wait_write(j1, 0)

        @pl.when(i < npair - 1)
        def _():
            issue_gather(j1 + 2, 0)

        wait_write(j2, 1)

        @pl.when(i < npair - 1)
        def _():
            issue_gather(j2 + 2, 1)

        return carry

    lax.fori_loop(0, npair, pair, 0, unroll=False)


def _stage2_spmem(tbl2, idx2, ne):
    ept = ne // NS
    nchunk = ept // _GC
    assert nchunk % 2 == 0
    mesh = plsc.VectorSubcoreMesh(core_axis_name="c", subcore_axis_name="s")
    fn = pl.kernel(
        functools.partial(_gather_spmem_body, ept, nchunk),
        out_type=jax.ShapeDtypeStruct((NC, ne, D), jnp.float32),
        mesh=mesh,
        scratch_types=[
            pltpu.VMEM_SHARED((N_MESH, D), jnp.float32),
            pltpu.VMEM((nchunk, _GC), jnp.int32),
            pltpu.VMEM((_GC, D), jnp.float32),
            pltpu.VMEM((_GC, D), jnp.float32),
            pltpu.SemaphoreType.DMA,
            pltpu.SemaphoreType.DMA,
            pltpu.SemaphoreType.DMA,
            pltpu.SemaphoreType.DMA,
        ],
    )
    return fn(tbl2, idx2)


def _stage2(pg, pm, src_r, dst_r, ne):
    epw = ne // NW
    nchunk = epw // CHUNK
    assert nchunk % 4 == 1 and nchunk >= 5
    mesh = plsc.VectorSubcoreMesh(core_axis_name="c", subcore_axis_name="s")
    fn = pl.kernel(
        functools.partial(_gather_body, epw, nchunk),
        out_type=jax.ShapeDtypeStruct((ne, D), jnp.float32),
        mesh=mesh,
        scratch_types=[
            pltpu.VMEM((nchunk, CHUNK), jnp.int32),
            pltpu.VMEM((nchunk, CHUNK), jnp.int32),
            [pltpu.VMEM((CHUNK, D), jnp.float32)] * _RING,
            [pltpu.VMEM((CHUNK, D), jnp.float32)] * _RING,
            [pltpu.SemaphoreType.DMA] * _RING,
            [pltpu.SemaphoreType.DMA] * _RING,
            [pltpu.SemaphoreType.DMA] * _RING,
        ],
    )
    return fn(pg, pm, src_r, dst_r)


def _edge_body(ef_ref, g2_ref, w0a_ref, w1_ref, eb1_ref, eg_ref, ebt_ref,
               out_ref):
    h = _silu(ef_ref[...] @ w0a_ref[...] + g2_ref[0] + g2_ref[1])
    y = h @ w1_ref[...] + eb1_ref[...]
    out_ref[...] = _ln(y, eg_ref[...], ebt_ref[...])


def _stage3(efeat, g2, w0a, w1, eb1, eg, ebt, blk0, ne):
    R = 1000
    erow = pl.BlockSpec((R, D), lambda i: (i + blk0, 0))
    row = pl.BlockSpec((R, D), lambda i: (i, 0))
    g2spec = pl.BlockSpec((NC, R, D), lambda i: (0, i, 0))
    mat = pl.BlockSpec((D, H), lambda i: (0, 0))
    vec = pl.BlockSpec((1, H), lambda i: (0, 0))
    return pl.pallas_call(
        _edge_body,
        grid=(ne // R,),
        in_specs=[erow, g2spec, mat, mat, vec, vec, vec],
        out_specs=row,
        out_shape=jax.ShapeDtypeStruct((ne, D), jnp.float32),
    )(efeat, g2, w0a, w1, eb1, eg, ebt)


def _scatter_body(epw, nchunk, nslice, *refs):
    y_hbms = refs[:nslice]
    (dst_hbm, init_hbm, out_hbm,
     idx_v, u0, u1, acc_sh, rs0, rs1, ss0, ss1) = refs[nslice:]
    c = lax.axis_index("c")
    s = lax.axis_index("s")
    w = s * NC + c
    base = w * epw

    @pl.when(s == 0)
    def _():
        pltpu.sync_copy(init_hbm.at[c], acc_sh)

    plsc.subcore_barrier()
    pltpu.sync_copy(dst_hbm.at[w], idx_v)
    bufs = ((u0, rs0, ss0), (u1, rs1, ss1))

    def run_slice(y_hbm, joff):
        def issue_read(j, p):
            u, rs, _ = bufs[p]
            pltpu.async_copy(y_hbm.at[pl.ds(base + j * CHUNK, CHUNK)], u, rs)

        def wait_read(j, p):
            u, rs, _ = bufs[p]
            pltpu.make_async_copy(
                y_hbm.at[pl.ds(base + j * CHUNK, CHUNK)], u, rs).wait()

        def issue_scatter(j, p):
            u, _, ss = bufs[p]
            pltpu.async_copy(u, acc_sh.at[idx_v.at[joff + j]], ss, add=True)

        def wait_scatter(j, p):
            u, _, ss = bufs[p]
            pltpu.make_async_copy(u, acc_sh.at[idx_v.at[joff + j]], ss).wait()

        npair = (nchunk - 1) // 2
        issue_read(0, 1)
        wait_read(0, 1)
        issue_scatter(0, 1)
        issue_read(1, 0)
        wait_scatter(0, 1)
        issue_read(2, 1)

        def pair(i, carry):
            j1 = 2 * i + 1
            j2 = 2 * i + 2
            wait_read(j1, 0)
            issue_scatter(j1, 0)
            wait_read(j2, 1)
            issue_scatter(j2, 1)
            wait_scatter(j1, 0)

            @pl.when(i < npair - 1)
            def _():
                issue_read(j1 + 2, 0)

            wait_scatter(j2, 1)

            @pl.when(i < npair - 1)
            def _():
                issue_read(j2 + 2, 1)

            return carry

        lax.fori_loop(0, npair, pair, 0, unroll=False)

    for si in range(nslice):
        run_slice(y_hbms[si], si * nchunk)
    plsc.subcore_barrier()

    @pl.when(s < 15)
    def _():
        pltpu.sync_copy(acc_sh.at[pl.ds(s * 640, 640)],
                        out_hbm.at[c, pl.ds(s * 640, 640)])

    @pl.when(s == 15)
    def _():
        pltpu.sync_copy(acc_sh.at[pl.ds(9600, 400)],
                        out_hbm.at[c, pl.ds(9600, 400)])


def _stage4(mlp_list, dst_r, init, ne):
    epw = ne // NW
    nchunk = epw // CHUNK
    nslice = len(mlp_list)
    mesh = plsc.VectorSubcoreMesh(core_axis_name="c", subcore_axis_name="s")
    fn = pl.kernel(
        functools.partial(_scatter_body, epw, nchunk, nslice),
        out_type=jax.ShapeDtypeStruct((NC, N_MESH, D), jnp.float32),
        mesh=mesh,
        scratch_types=[
            pltpu.VMEM((nslice * nchunk, CHUNK), jnp.int32),
            pltpu.VMEM((CHUNK, D), jnp.float32),
            pltpu.VMEM((CHUNK, D), jnp.float32),
            pltpu.VMEM_SHARED((N_MESH, D), jnp.float32),
            pltpu.SemaphoreType.DMA,
            pltpu.SemaphoreType.DMA,
            pltpu.SemaphoreType.DMA,
            pltpu.SemaphoreType.DMA,
        ],
    )
    return fn(*mlp_list, dst_r, init)


def _s5_body(*refs):
    nparts = len(refs) - 9
    parts = refs[:nparts]
    (mesh_ref, dw0a_ref, dw0b_ref, db0_ref,
     dw1_ref, db1_ref, dg_ref, dbt_ref, out_ref) = refs[nparts:]
    agg = parts[0][0] + parts[0][1]
    for p in parts[1:]:
        agg = agg + p[0] + p[1]
    m = mesh_ref[...]
    h = _silu(agg @ dw0a_ref[...] + m @ dw0b_ref[...] + db0_ref[...])
    y = h @ dw1_ref[...] + db1_ref[...]
    out_ref[...] = m + _ln(y, dg_ref[...], dbt_ref[...])


def _stage5(parts, mesh_nfeat, dw0a, dw0b, db0, dw1, db1, dg, dbt):
    R = 1000
    row = pl.BlockSpec((R, D), lambda i: (i, 0))
    mat = pl.BlockSpec((D, H), lambda i: (0, 0))
    vec = pl.BlockSpec((1, H), lambda i: (0, 0))
    pspec = pl.BlockSpec((NC, R, D), lambda i: (0, i, 0))
    return pl.pallas_call(
        _s5_body,
        grid=(N_MESH // R,),
        in_specs=[pspec] * len(parts) + [row, mat, mat, vec, mat, vec, vec, vec],
        out_specs=row,
        out_shape=jax.ShapeDtypeStruct((N_MESH, D), jnp.float32),
    )(*parts, mesh_nfeat, dw0a, dw0b, db0, dw1, db1, dg, dbt)


def kernel(g2m_efeat, grid_nfeat, mesh_nfeat,
           eW0, eb0, eW1, eb1, eg, ebt,
           sW0, sb0, sW1, sb1, sg, sbt,
           dW0, db0, dW1, db1, dg, dbt,
           src, dst):
    w0a, w0b, w0c = eW0[:D], eW0[D:2 * D], eW0[2 * D:]
    dw0a, dw0b = dW0[:D], dW0[D:]
    r2 = lambda v: v.reshape(1, -1)

    tbl2 = _stage1a(grid_nfeat, mesh_nfeat, w0b, w0c, r2(eb0))
    grid_out = _stage1b(grid_nfeat, sW0, r2(sb0), sW1, r2(sb1), r2(sg),
                        r2(sbt))

    S = 5
    ES = E // S
    nchunk = ES // NW // CHUNK
    dst_r = dst.reshape(S, NW, nchunk, CHUNK)
    ngc = ES // NS // _GC
    idx_r = jnp.stack([src, dst]).reshape(2, S, NS, ngc, _GC).transpose(
        1, 0, 2, 3, 4)

    part = jnp.zeros((NC, N_MESH, D), jnp.float32)
    for i in range(S):
        g2 = _stage2_spmem(tbl2, idx_r[i], ES)
        mlp_e = _stage3(g2m_efeat, g2, w0a, eW1, r2(eb1), r2(eg), r2(ebt),
                        i * (ES // 1000), ES)
        part = _stage4([mlp_e], dst_r[i], part, ES)

    mesh_out = _stage5([part], mesh_nfeat, dw0a, dw0b, r2(db0),
                       dW1, r2(db1), r2(dg), r2(dbt))
    return (grid_out, mesh_out)
